# baseline (speedup 1.0000x reference)
import jax
import jax.numpy as jnp
from jax.experimental import pallas as pl
from jax.experimental.pallas import tpu as pltpu


def _gcn_body(adj_ref, nodes_all_ref, nodes_ref, w0_ref, b0_ref, wo_ref,
              bo_ref, out_ref):
    a = adj_ref[0]
    denom = jnp.sum(a, axis=1, keepdims=True) + 1.0
    h = jnp.dot(
        a.astype(jnp.bfloat16),
        nodes_all_ref[0].astype(jnp.bfloat16),
        preferred_element_type=jnp.float32,
    ) + nodes_ref[0]
    hw = jnp.dot(h, w0_ref[...], preferred_element_type=jnp.float32)
    g = jnp.maximum(hw / denom + b0_ref[0], 0.0) + nodes_ref[0]
    out_ref[0] = (
        jnp.dot(g, wo_ref[...], preferred_element_type=jnp.float32) + bo_ref[0]
    )


def kernel(nodes, adj, W0, b0, Wo, bo):
    B, N, D = nodes.shape
    BN = 2048

    return pl.pallas_call(
        _gcn_body,
        grid=(B, N // BN),
        in_specs=[
            pl.BlockSpec((1, BN, N), lambda b, i: (b, i, 0)),
            pl.BlockSpec((1, N, D), lambda b, i: (b, 0, 0)),
            pl.BlockSpec((1, BN, D), lambda b, i: (b, i, 0)),
            pl.BlockSpec((D, D), lambda b, i: (0, 0)),
            pl.BlockSpec((1, D), lambda b, i: (0, 0)),
            pl.BlockSpec((D, D), lambda b, i: (0, 0)),
            pl.BlockSpec((1, D), lambda b, i: (0, 0)),
        ],
        out_specs=pl.BlockSpec((1, BN, D), lambda b, i: (b, i, 0)),
        out_shape=jax.ShapeDtypeStruct((B, N, D), jnp.float32),
        compiler_params=pltpu.CompilerParams(
            dimension_semantics=("parallel", "parallel"),
        ),
    )(adj, nodes, nodes, W0, b0.reshape(1, D), Wo, bo.reshape(1, D))


# speedup vs baseline: 1.0565x; 1.0565x over previous
import functools

import jax
import jax.numpy as jnp
from jax.experimental import pallas as pl
from jax.experimental.pallas import tpu as pltpu


def _gcn_body(bn, adj_ref, nodes_all_ref, w0_ref, b0_ref, wo_ref,
              bo_ref, out_ref):
    i = pl.program_id(1)
    a = adj_ref[0]
    denom = jnp.sum(a, axis=1, keepdims=True) + 1.0
    nodes_blk = nodes_all_ref[0, pl.ds(i * bn, bn), :]
    h = jnp.dot(
        a.astype(jnp.bfloat16),
        nodes_all_ref[0].astype(jnp.bfloat16),
        preferred_element_type=jnp.float32,
    ) + nodes_blk
    hw = jnp.dot(h, w0_ref[...], preferred_element_type=jnp.float32)
    g = jnp.maximum(hw / denom + b0_ref[0], 0.0) + nodes_blk
    out_ref[0] = (
        jnp.dot(g, wo_ref[...], preferred_element_type=jnp.float32) + bo_ref[0]
    )


def kernel(nodes, adj, W0, b0, Wo, bo):
    B, N, D = nodes.shape
    BN = 1024

    return pl.pallas_call(
        functools.partial(_gcn_body, BN),
        grid=(B, N // BN),
        in_specs=[
            pl.BlockSpec((1, BN, N), lambda b, i: (b, i, 0)),
            pl.BlockSpec((1, N, D), lambda b, i: (b, 0, 0)),
            pl.BlockSpec((D, D), lambda b, i: (0, 0)),
            pl.BlockSpec((1, D), lambda b, i: (0, 0)),
            pl.BlockSpec((D, D), lambda b, i: (0, 0)),
            pl.BlockSpec((1, D), lambda b, i: (0, 0)),
        ],
        out_specs=pl.BlockSpec((1, BN, D), lambda b, i: (b, i, 0)),
        out_shape=jax.ShapeDtypeStruct((B, N, D), jnp.float32),
        compiler_params=pltpu.CompilerParams(
            dimension_semantics=("parallel", "parallel"),
        ),
    )(adj, nodes, W0, b0.reshape(1, D), Wo, bo.reshape(1, D))
